# Initial kernel scaffold; baseline (speedup 1.0000x reference)
import jax
import jax.numpy as jnp
from jax.experimental import pallas as pl


def kernel(x, W_in, b_in, W_router, b_router, W_experts, b_experts, W_out, b_out):
    raise NotImplementedError("write your pallas kernel here")


# speedup vs baseline: 2.3172x; 2.3172x over previous
import functools

import jax
import jax.numpy as jnp
from jax.experimental import pallas as pl

IN_FEATURES = 768
OUT_FEATURES = 768
HIDDEN = 64
NUM_EXPERTS = 64
MAX_ROUTING = 4
SSF = 0.95
N_TOK = 8192

BT = 256


_HI = jax.lax.Precision.HIGHEST


def _rmlp_block(x_ref, w_in_ref, b_in_ref, w_router_ref, b_router_ref,
                w_flat_ref, b_exp_ref, w_out_ref, b_out_ref, out_ref):
    f32 = jnp.float32
    x = x_ref[...]
    h = jnp.maximum(
        jnp.dot(x, w_in_ref[...], preferred_element_type=f32,
                ) + b_in_ref[...], 0.0)

    lane_e = jax.lax.broadcasted_iota(jnp.int32, (BT, NUM_EXPERTS), 1)
    lane_big = jax.lax.broadcasted_iota(jnp.int32, (BT, NUM_EXPERTS * HIDDEN), 1)
    grp_big = lane_big // HIDDEN

    scale = 1.0
    for _ in range(MAX_ROUTING):
        logits = jnp.dot(h, w_router_ref[...], preferred_element_type=f32,
) + b_router_ref[...]
        m = jnp.max(logits, axis=-1, keepdims=True)
        s = jnp.sum(jnp.exp(logits - m), axis=-1, keepdims=True)
        gate = 1.0 / (1.0 + 1e-9 * s)
        idx = jnp.min(jnp.where(logits == m, lane_e, NUM_EXPERTS),
                      axis=-1, keepdims=True)
        onehot = (lane_e == idx).astype(f32)
        big = jnp.tile(h, (1, NUM_EXPERTS))
        M = jnp.where(grp_big == idx, big, 0.0)
        b_sel = jnp.dot(onehot, b_exp_ref[...], preferred_element_type=f32,
)
        eo = jnp.maximum(
            jnp.dot(M, w_flat_ref[...], preferred_element_type=f32,
) + b_sel, 0.0)
        h = eo * (gate * scale)
        scale = scale * SSF

    out_ref[...] = jnp.dot(h, w_out_ref[...], preferred_element_type=f32) \
        + b_out_ref[...]


@jax.jit
def kernel(x, W_in, b_in, W_router, b_router, W_experts, b_experts, W_out, b_out):
    x = x.reshape(x.shape[0], -1)
    n = x.shape[0]
    w_flat = W_experts.reshape(NUM_EXPERTS * HIDDEN, HIDDEN)

    full = lambda shape: pl.BlockSpec(shape, lambda i: (0,) * len(shape))
    grid = (n // BT,)
    out = pl.pallas_call(
        _rmlp_block,
        grid=grid,
        in_specs=[
            pl.BlockSpec((BT, IN_FEATURES), lambda i: (i, 0)),
            full((IN_FEATURES, HIDDEN)),
            full((HIDDEN,)),
            full((HIDDEN, NUM_EXPERTS)),
            full((NUM_EXPERTS,)),
            full((NUM_EXPERTS * HIDDEN, HIDDEN)),
            full((NUM_EXPERTS, HIDDEN)),
            full((HIDDEN, OUT_FEATURES)),
            full((OUT_FEATURES,)),
        ],
        out_specs=pl.BlockSpec((BT, OUT_FEATURES), lambda i: (i, 0)),
        out_shape=jax.ShapeDtypeStruct((n, OUT_FEATURES), jnp.float32),
    )(x, W_in, b_in, W_router, b_router, w_flat, b_experts, W_out, b_out)
    return out
